# baseline (speedup 1.0000x reference)
import functools

import jax
import jax.numpy as jnp
from jax import lax
from jax.experimental import pallas as pl
from jax.experimental.pallas import tpu as pltpu
from jax.experimental.pallas import tpu_sc as plsc

NUM_TILES = 4
BN = 512
NW = 32
SC_NC = 2
SC_NS = 16
CH = 16


def _sigs_body(up_ref, sigs_ref, ssc_ref, *, rows, tile, nsteps):
    k = pl.program_id(0)
    part = jnp.sum(jnp.sign(up_ref[...]), axis=0, keepdims=True)
    blocks_per_tile = tile // rows
    row = k // blocks_per_tile

    @pl.when(k % blocks_per_tile == 0)
    def _():
        ssc_ref[pl.ds(row, 1), :] = part

    @pl.when(k % blocks_per_tile != 0)
    def _():
        ssc_ref[pl.ds(row, 1), :] += part

    @pl.when(k == nsteps - 1)
    def _():
        mean = ssc_ref[...] * (1.0 / tile)
        nrm = jnp.sqrt(jnp.sum(mean * mean, axis=1, keepdims=True))
        sigs_ref[...] = mean / (nrm + 1e-8)


def _cast_body(up_ref, dn_ref, uw_ref, dw_ref):
    uw_ref[...] = jnp.sign(up_ref[...]).astype(jnp.bfloat16)
    dw_ref[...] = jnp.sign(dn_ref[...]).astype(jnp.bfloat16)


def _router_body(x_ref, sigs_ref, gate_ref, pos_ref, bt_ref, gsc_ref,
                 *, bn, n, nb):
    i = pl.program_id(0)
    xb = x_ref[...].astype(jnp.bfloat16)
    s = lax.dot_general(xb, sigs_ref[...].astype(jnp.bfloat16),
                        (((1,), (1,)), ((), ())),
                        preferred_element_type=jnp.float32)
    m = jnp.max(s, axis=1, keepdims=True)
    t_iota = lax.broadcasted_iota(jnp.int32, (bn, NUM_TILES), 1)
    cand = jnp.where(s == m, t_iota, NUM_TILES)
    winner = jnp.min(cand, axis=1, keepdims=True)
    gate = (t_iota == winner).astype(jnp.float32)
    gate_ref[...] = gate
    gsc_ref[pl.ds(i * bn, bn), :] = gate

    @pl.when(i == (n // bn) - 1)
    def _():
        g = gsc_ref[...]
        c = g
        sh = 1
        while sh < n:
            c = c + jnp.concatenate(
                [jnp.zeros((sh, NUM_TILES), jnp.float32), c[:-sh, :]],
                axis=0)
            sh *= 2
        rank = c - g
        counts = c[n - 1:n, :]
        pc = jnp.floor((counts + (BN - 1.0)) / BN) * BN
        p1 = pc[:, 0:1]
        p2 = p1 + pc[:, 1:2]
        p3 = p2 + pc[:, 2:3]
        pb = jnp.concatenate(
            [jnp.zeros((1, 1), jnp.float32), p1, p2, p3], axis=1)
        pos = jnp.sum(g * (rank + pb), axis=1, keepdims=True)
        pos_ref[...] = pos.astype(jnp.int32)
        ends = pb + pc
        bidx = lax.broadcasted_iota(jnp.int32, (nb, NUM_TILES), 0).astype(
            jnp.float32) * BN
        btv = jnp.sum((bidx >= ends).astype(jnp.float32), axis=1,
                      keepdims=True)
        bt_ref[...] = jnp.minimum(btv, NUM_TILES - 1.0).astype(jnp.int32)


def _ffn_body(bt_ref, xs_ref, uw_ref, usc_ref, dw_ref, dsc_ref, out_ref):
    del bt_ref
    xb = xs_ref[...].astype(jnp.bfloat16)
    h = lax.dot_general(xb, uw_ref[...], (((1,), (1,)), ((), ())),
                        preferred_element_type=jnp.float32)
    h = jnp.maximum(h * usc_ref[...], 0.0).astype(jnp.bfloat16)
    o = lax.dot_general(h, dw_ref[...], (((1,), (1,)), ((), ())),
                        preferred_element_type=jnp.float32)
    out_ref[...] = o * dsc_ref[...]


def _make_sc_scatter(n, n_pad, dp):
    per_w = n // NW
    j_s = per_w // CH
    mesh = plsc.VectorSubcoreMesh(core_axis_name="c", subcore_axis_name="s",
                                  num_cores=SC_NC, num_subcores=SC_NS)

    @functools.partial(
        pl.kernel, mesh=mesh,
        out_type=jax.ShapeDtypeStruct((n_pad, dp), jnp.float32),
        scratch_types=[
            pltpu.VMEM((CH, dp), jnp.float32),
            pltpu.VMEM((CH, dp), jnp.float32),
            pltpu.VMEM((j_s, CH), jnp.int32),
            pltpu.SemaphoreType.DMA,
            pltpu.SemaphoreType.DMA,
            pltpu.SemaphoreType.DMA,
            pltpu.SemaphoreType.DMA,
        ],
    )
    def scatter_k(x_hbm, pos3_hbm, xs_hbm, buf0, buf1, posv,
                  rs0, rs1, ws0, ws1):
        wid = lax.axis_index("s") * SC_NC + lax.axis_index("c")
        base = wid * per_w
        pltpu.sync_copy(pos3_hbm.at[wid], posv)
        bufs = (buf0, buf1)
        rsems = (rs0, rs1)
        wsems = (ws0, ws1)
        rds = [None, None]
        wrs = [None, None]
        rds[0] = pltpu.async_copy(x_hbm.at[pl.ds(base, CH)], bufs[0], rsems[0])
        for j in range(j_s):
            bsel = j & 1
            osel = bsel ^ 1
            if wrs[osel] is not None:
                wrs[osel].wait()
                wrs[osel] = None
            if j + 1 < j_s:
                rds[osel] = pltpu.async_copy(
                    x_hbm.at[pl.ds(base + (j + 1) * CH, CH)],
                    bufs[osel], rsems[osel])
            rds[bsel].wait()
            wrs[bsel] = pltpu.async_copy(
                bufs[bsel], xs_hbm.at[posv.at[j]], wsems[bsel])
        for w in wrs:
            if w is not None:
                w.wait()

    return scatter_k


def _make_sc_gather(n, n_pad, d_model):
    per_w = n // NW
    j_g = per_w // CH
    mesh = plsc.VectorSubcoreMesh(core_axis_name="c", subcore_axis_name="s",
                                  num_cores=SC_NC, num_subcores=SC_NS)

    @functools.partial(
        pl.kernel, mesh=mesh,
        out_type=jax.ShapeDtypeStruct((n, d_model), jnp.float32),
        scratch_types=[
            pltpu.VMEM((CH, d_model), jnp.float32),
            pltpu.VMEM((CH, d_model), jnp.float32),
            pltpu.VMEM((j_g, CH), jnp.int32),
            pltpu.SemaphoreType.DMA,
            pltpu.SemaphoreType.DMA,
            pltpu.SemaphoreType.DMA,
            pltpu.SemaphoreType.DMA,
        ],
    )
    def gather_k(outs_hbm, pos3_hbm, out_hbm, buf0, buf1, posv,
                 rs0, rs1, ws0, ws1):
        wid = lax.axis_index("s") * SC_NC + lax.axis_index("c")
        base = wid * per_w
        pltpu.sync_copy(pos3_hbm.at[wid], posv)
        bufs = (buf0, buf1)
        rsems = (rs0, rs1)
        wsems = (ws0, ws1)
        rds = [None, None]
        wrs = [None, None]
        rds[0] = pltpu.async_copy(outs_hbm.at[posv.at[0]], bufs[0], rsems[0])
        for j in range(j_g):
            bsel = j & 1
            osel = bsel ^ 1
            if wrs[osel] is not None:
                wrs[osel].wait()
                wrs[osel] = None
            if j + 1 < j_g:
                rds[osel] = pltpu.async_copy(
                    outs_hbm.at[posv.at[j + 1]], bufs[osel], rsems[osel])
            rds[bsel].wait()
            wrs[bsel] = pltpu.async_copy(
                bufs[bsel], out_hbm.at[pl.ds(base + j * CH, CH)],
                wsems[bsel])
        for w in wrs:
            if w is not None:
                w.wait()

    return gather_k


def kernel(x, up_w, up_scales, down_w, down_scales):
    b, t, d_model = x.shape
    d_ff = up_w.shape[0]
    tile = d_ff // NUM_TILES
    n = b * t
    n_pad = n + NUM_TILES * BN
    nb = n_pad // BN

    xf = x.reshape(n, d_model)
    usc2 = up_scales.reshape(1, d_ff)
    dsc = down_scales.reshape(1, d_model)

    rows0 = min(512, tile)
    nst0 = d_ff // rows0
    sigs = pl.pallas_call(
        functools.partial(_sigs_body, rows=rows0, tile=tile, nsteps=nst0),
        grid=(nst0,),
        in_specs=[pl.BlockSpec((rows0, d_model), lambda k: (k, 0))],
        out_specs=pl.BlockSpec((NUM_TILES, d_model), lambda k: (0, 0)),
        out_shape=jax.ShapeDtypeStruct((NUM_TILES, d_model), jnp.float32),
        scratch_shapes=[pltpu.VMEM((NUM_TILES, d_model), jnp.float32)],
        compiler_params=pltpu.CompilerParams(
            dimension_semantics=("arbitrary",),
        ),
    )(up_w)

    bn_r = min(512, n)
    gate, pos, bt = pl.pallas_call(
        functools.partial(_router_body, bn=bn_r, n=n, nb=nb),
        grid=(n // bn_r,),
        in_specs=[
            pl.BlockSpec((bn_r, d_model), lambda i: (i, 0)),
            pl.BlockSpec((NUM_TILES, d_model), lambda i: (0, 0)),
        ],
        out_specs=[
            pl.BlockSpec((bn_r, NUM_TILES), lambda i: (i, 0)),
            pl.BlockSpec((n, 1), lambda i: (0, 0)),
            pl.BlockSpec((nb, 1), lambda i: (0, 0)),
        ],
        out_shape=[
            jax.ShapeDtypeStruct((n, NUM_TILES), jnp.float32),
            jax.ShapeDtypeStruct((n, 1), jnp.int32),
            jax.ShapeDtypeStruct((nb, 1), jnp.int32),
        ],
        scratch_shapes=[pltpu.VMEM((n, NUM_TILES), jnp.float32)],
    )(xf, sigs)
    posf = pos.reshape(n)
    btf = bt.reshape(nb)

    pos3 = posf.reshape(NW, n // NW // CH, CH)
    xs = _make_sc_scatter(n, n_pad, d_model)(xf, pos3)

    rows = min(512, tile)
    nsteps = d_ff // rows
    uw2, dw2 = pl.pallas_call(
        _cast_body,
        grid=(nsteps,),
        in_specs=[
            pl.BlockSpec((rows, d_model), lambda k: (k, 0)),
            pl.BlockSpec((d_model, rows), lambda k: (0, k)),
        ],
        out_specs=[
            pl.BlockSpec((rows, d_model), lambda k: (k, 0)),
            pl.BlockSpec((d_model, rows), lambda k: (0, k)),
        ],
        out_shape=[
            jax.ShapeDtypeStruct((d_ff, d_model), jnp.bfloat16),
            jax.ShapeDtypeStruct((d_model, d_ff), jnp.bfloat16),
        ],
        compiler_params=pltpu.CompilerParams(
            dimension_semantics=("arbitrary",),
        ),
    )(up_w, down_w)

    grid_spec = pltpu.PrefetchScalarGridSpec(
        num_scalar_prefetch=1,
        grid=(nb,),
        in_specs=[
            pl.BlockSpec((BN, d_model), lambda i, bt_r: (i, 0)),
            pl.BlockSpec((tile, d_model), lambda i, bt_r: (bt_r[i], 0)),
            pl.BlockSpec((1, tile), lambda i, bt_r: (0, bt_r[i])),
            pl.BlockSpec((d_model, tile), lambda i, bt_r: (0, bt_r[i])),
            pl.BlockSpec((1, d_model), lambda i, bt_r: (0, 0)),
        ],
        out_specs=pl.BlockSpec((BN, d_model), lambda i, bt_r: (i, 0)),
    )
    outs = pl.pallas_call(
        _ffn_body,
        grid_spec=grid_spec,
        out_shape=jax.ShapeDtypeStruct((n_pad, d_model), jnp.float32),
        compiler_params=pltpu.CompilerParams(
            dimension_semantics=("arbitrary",),
        ),
    )(btf, xs, uw2, usc2, dw2, dsc)

    out = _make_sc_gather(n, n_pad, d_model)(outs, pos3)

    return out.reshape(b, t, d_model), gate.reshape(b, t, NUM_TILES)


# speedup vs baseline: 1.0877x; 1.0877x over previous
import functools

import jax
import jax.numpy as jnp
from jax import lax
from jax.experimental import pallas as pl
from jax.experimental.pallas import tpu as pltpu
from jax.experimental.pallas import tpu_sc as plsc

NUM_TILES = 4
BN = 512
NW = 32
SC_NC = 2
SC_NS = 16
CH = 16


def _prep_body(up_ref, dn_ref, uw_ref, dw_ref, sigs_ref, ssc_ref,
               *, rows, tile, nsteps):
    k = pl.program_id(0)
    su = jnp.sign(up_ref[...])
    uw_ref[...] = su.astype(jnp.bfloat16)
    dw_ref[...] = jnp.sign(dn_ref[...]).astype(jnp.bfloat16)
    part = jnp.sum(su, axis=0, keepdims=True)
    blocks_per_tile = tile // rows
    row = k // blocks_per_tile

    @pl.when(k % blocks_per_tile == 0)
    def _():
        ssc_ref[pl.ds(row, 1), :] = part

    @pl.when(k % blocks_per_tile != 0)
    def _():
        ssc_ref[pl.ds(row, 1), :] += part

    @pl.when(k == nsteps - 1)
    def _():
        mean = ssc_ref[...] * (1.0 / tile)
        nrm = jnp.sqrt(jnp.sum(mean * mean, axis=1, keepdims=True))
        sigs_ref[...] = mean / (nrm + 1e-8)


def _router_body(x_ref, sigs_ref, gate_ref, pos_ref, bt_ref, gsc_ref,
                 *, bn, n, nb):
    i = pl.program_id(0)
    xb = x_ref[...].astype(jnp.bfloat16)
    s = lax.dot_general(xb, sigs_ref[...].astype(jnp.bfloat16),
                        (((1,), (1,)), ((), ())),
                        preferred_element_type=jnp.float32)
    m = jnp.max(s, axis=1, keepdims=True)
    t_iota = lax.broadcasted_iota(jnp.int32, (bn, NUM_TILES), 1)
    cand = jnp.where(s == m, t_iota, NUM_TILES)
    winner = jnp.min(cand, axis=1, keepdims=True)
    gate = (t_iota == winner).astype(jnp.float32)
    gate_ref[...] = gate
    gsc_ref[pl.ds(i * bn, bn), :] = gate

    @pl.when(i == (n // bn) - 1)
    def _():
        g = gsc_ref[...]
        c = g
        sh = 1
        while sh < n:
            c = c + jnp.concatenate(
                [jnp.zeros((sh, NUM_TILES), jnp.float32), c[:-sh, :]],
                axis=0)
            sh *= 2
        rank = c - g
        counts = c[n - 1:n, :]
        pc = jnp.floor((counts + (BN - 1.0)) / BN) * BN
        p1 = pc[:, 0:1]
        p2 = p1 + pc[:, 1:2]
        p3 = p2 + pc[:, 2:3]
        pb = jnp.concatenate(
            [jnp.zeros((1, 1), jnp.float32), p1, p2, p3], axis=1)
        pos = jnp.sum(g * (rank + pb), axis=1, keepdims=True)
        pos_ref[...] = pos.astype(jnp.int32)
        ends = pb + pc
        bidx = lax.broadcasted_iota(jnp.int32, (nb, NUM_TILES), 0).astype(
            jnp.float32) * BN
        btv = jnp.sum((bidx >= ends).astype(jnp.float32), axis=1,
                      keepdims=True)
        bt_ref[...] = jnp.minimum(btv, NUM_TILES - 1.0).astype(jnp.int32)


def _ffn_body(bt_ref, xs_ref, uw_ref, usc_ref, dw_ref, dsc_ref, out_ref):
    del bt_ref
    xb = xs_ref[...].astype(jnp.bfloat16)
    h = lax.dot_general(xb, uw_ref[...], (((1,), (1,)), ((), ())),
                        preferred_element_type=jnp.float32)
    h = jnp.maximum(h * usc_ref[...], 0.0).astype(jnp.bfloat16)
    o = lax.dot_general(h, dw_ref[...], (((1,), (1,)), ((), ())),
                        preferred_element_type=jnp.float32)
    out_ref[...] = o * dsc_ref[...]


def _make_sc_scatter(n, n_pad, dp):
    per_w = n // NW
    j_s = per_w // CH
    mesh = plsc.VectorSubcoreMesh(core_axis_name="c", subcore_axis_name="s",
                                  num_cores=SC_NC, num_subcores=SC_NS)

    @functools.partial(
        pl.kernel, mesh=mesh,
        out_type=jax.ShapeDtypeStruct((n_pad, dp), jnp.float32),
        scratch_types=[
            pltpu.VMEM((CH, dp), jnp.float32),
            pltpu.VMEM((CH, dp), jnp.float32),
            pltpu.VMEM((j_s, CH), jnp.int32),
            pltpu.SemaphoreType.DMA,
            pltpu.SemaphoreType.DMA,
            pltpu.SemaphoreType.DMA,
            pltpu.SemaphoreType.DMA,
        ],
    )
    def scatter_k(x_hbm, pos3_hbm, xs_hbm, buf0, buf1, posv,
                  rs0, rs1, ws0, ws1):
        wid = lax.axis_index("s") * SC_NC + lax.axis_index("c")
        base = wid * per_w
        pltpu.sync_copy(pos3_hbm.at[wid], posv)
        bufs = (buf0, buf1)
        rsems = (rs0, rs1)
        wsems = (ws0, ws1)
        rds = [None, None]
        wrs = [None, None]
        rds[0] = pltpu.async_copy(x_hbm.at[pl.ds(base, CH)], bufs[0], rsems[0])
        for j in range(j_s):
            bsel = j & 1
            osel = bsel ^ 1
            if wrs[osel] is not None:
                wrs[osel].wait()
                wrs[osel] = None
            if j + 1 < j_s:
                rds[osel] = pltpu.async_copy(
                    x_hbm.at[pl.ds(base + (j + 1) * CH, CH)],
                    bufs[osel], rsems[osel])
            rds[bsel].wait()
            wrs[bsel] = pltpu.async_copy(
                bufs[bsel], xs_hbm.at[posv.at[j]], wsems[bsel])
        for w in wrs:
            if w is not None:
                w.wait()

    return scatter_k


def _make_sc_gather(n, n_pad, d_model):
    per_w = n // NW
    j_g = per_w // CH
    mesh = plsc.VectorSubcoreMesh(core_axis_name="c", subcore_axis_name="s",
                                  num_cores=SC_NC, num_subcores=SC_NS)

    @functools.partial(
        pl.kernel, mesh=mesh,
        out_type=jax.ShapeDtypeStruct((n, d_model), jnp.float32),
        scratch_types=[
            pltpu.VMEM((CH, d_model), jnp.float32),
            pltpu.VMEM((CH, d_model), jnp.float32),
            pltpu.VMEM((j_g, CH), jnp.int32),
            pltpu.SemaphoreType.DMA,
            pltpu.SemaphoreType.DMA,
            pltpu.SemaphoreType.DMA,
            pltpu.SemaphoreType.DMA,
        ],
    )
    def gather_k(outs_hbm, pos3_hbm, out_hbm, buf0, buf1, posv,
                 rs0, rs1, ws0, ws1):
        wid = lax.axis_index("s") * SC_NC + lax.axis_index("c")
        base = wid * per_w
        pltpu.sync_copy(pos3_hbm.at[wid], posv)
        bufs = (buf0, buf1)
        rsems = (rs0, rs1)
        wsems = (ws0, ws1)
        rds = [None, None]
        wrs = [None, None]
        rds[0] = pltpu.async_copy(outs_hbm.at[posv.at[0]], bufs[0], rsems[0])
        for j in range(j_g):
            bsel = j & 1
            osel = bsel ^ 1
            if wrs[osel] is not None:
                wrs[osel].wait()
                wrs[osel] = None
            if j + 1 < j_g:
                rds[osel] = pltpu.async_copy(
                    outs_hbm.at[posv.at[j + 1]], bufs[osel], rsems[osel])
            rds[bsel].wait()
            wrs[bsel] = pltpu.async_copy(
                bufs[bsel], out_hbm.at[pl.ds(base + j * CH, CH)],
                wsems[bsel])
        for w in wrs:
            if w is not None:
                w.wait()

    return gather_k


def kernel(x, up_w, up_scales, down_w, down_scales):
    b, t, d_model = x.shape
    d_ff = up_w.shape[0]
    tile = d_ff // NUM_TILES
    n = b * t
    n_pad = n + NUM_TILES * BN
    nb = n_pad // BN

    xf = x.reshape(n, d_model)
    usc2 = up_scales.reshape(1, d_ff)
    dsc = down_scales.reshape(1, d_model)

    rows = min(512, tile)
    nsteps = d_ff // rows
    uw2, dw2, sigs = pl.pallas_call(
        functools.partial(_prep_body, rows=rows, tile=tile, nsteps=nsteps),
        grid=(nsteps,),
        in_specs=[
            pl.BlockSpec((rows, d_model), lambda k: (k, 0)),
            pl.BlockSpec((d_model, rows), lambda k: (0, k)),
        ],
        out_specs=[
            pl.BlockSpec((rows, d_model), lambda k: (k, 0)),
            pl.BlockSpec((d_model, rows), lambda k: (0, k)),
            pl.BlockSpec((NUM_TILES, d_model), lambda k: (0, 0)),
        ],
        out_shape=[
            jax.ShapeDtypeStruct((d_ff, d_model), jnp.bfloat16),
            jax.ShapeDtypeStruct((d_model, d_ff), jnp.bfloat16),
            jax.ShapeDtypeStruct((NUM_TILES, d_model), jnp.float32),
        ],
        scratch_shapes=[pltpu.VMEM((NUM_TILES, d_model), jnp.float32)],
        compiler_params=pltpu.CompilerParams(
            dimension_semantics=("arbitrary",),
        ),
    )(up_w, down_w)

    bn_r = min(512, n)
    gate, pos, bt = pl.pallas_call(
        functools.partial(_router_body, bn=bn_r, n=n, nb=nb),
        grid=(n // bn_r,),
        in_specs=[
            pl.BlockSpec((bn_r, d_model), lambda i: (i, 0)),
            pl.BlockSpec((NUM_TILES, d_model), lambda i: (0, 0)),
        ],
        out_specs=[
            pl.BlockSpec((bn_r, NUM_TILES), lambda i: (i, 0)),
            pl.BlockSpec((n, 1), lambda i: (0, 0)),
            pl.BlockSpec((nb, 1), lambda i: (0, 0)),
        ],
        out_shape=[
            jax.ShapeDtypeStruct((n, NUM_TILES), jnp.float32),
            jax.ShapeDtypeStruct((n, 1), jnp.int32),
            jax.ShapeDtypeStruct((nb, 1), jnp.int32),
        ],
        scratch_shapes=[pltpu.VMEM((n, NUM_TILES), jnp.float32)],
    )(xf, sigs)
    posf = pos.reshape(n)
    btf = bt.reshape(nb)

    pos3 = posf.reshape(NW, n // NW // CH, CH)
    xs = _make_sc_scatter(n, n_pad, d_model)(xf, pos3)

    grid_spec = pltpu.PrefetchScalarGridSpec(
        num_scalar_prefetch=1,
        grid=(nb,),
        in_specs=[
            pl.BlockSpec((BN, d_model), lambda i, bt_r: (i, 0)),
            pl.BlockSpec((tile, d_model), lambda i, bt_r: (bt_r[i], 0)),
            pl.BlockSpec((1, tile), lambda i, bt_r: (0, bt_r[i])),
            pl.BlockSpec((d_model, tile), lambda i, bt_r: (0, bt_r[i])),
            pl.BlockSpec((1, d_model), lambda i, bt_r: (0, 0)),
        ],
        out_specs=pl.BlockSpec((BN, d_model), lambda i, bt_r: (i, 0)),
    )
    outs = pl.pallas_call(
        _ffn_body,
        grid_spec=grid_spec,
        out_shape=jax.ShapeDtypeStruct((n_pad, d_model), jnp.float32),
        compiler_params=pltpu.CompilerParams(
            dimension_semantics=("arbitrary",),
        ),
    )(btf, xs, uw2, usc2, dw2, dsc)

    out = _make_sc_gather(n, n_pad, d_model)(outs, pos3)

    return out.reshape(b, t, d_model), gate.reshape(b, t, NUM_TILES)


# speedup vs baseline: 1.0888x; 1.0010x over previous
import functools

import jax
import jax.numpy as jnp
from jax import lax
from jax.experimental import pallas as pl
from jax.experimental.pallas import tpu as pltpu
from jax.experimental.pallas import tpu_sc as plsc

NUM_TILES = 4
BN = 256
NW = 32
SC_NC = 2
SC_NS = 16
CH = 16


def _prep_body(up_ref, dn_ref, uw_ref, dw_ref, sigs_ref, ssc_ref,
               *, rows, tile, nsteps):
    k = pl.program_id(0)
    su = jnp.sign(up_ref[...])
    uw_ref[...] = su.astype(jnp.bfloat16)
    dw_ref[...] = jnp.sign(dn_ref[...]).astype(jnp.bfloat16)
    part = jnp.sum(su, axis=0, keepdims=True)
    blocks_per_tile = tile // rows
    row = k // blocks_per_tile

    @pl.when(k % blocks_per_tile == 0)
    def _():
        ssc_ref[pl.ds(row, 1), :] = part

    @pl.when(k % blocks_per_tile != 0)
    def _():
        ssc_ref[pl.ds(row, 1), :] += part

    @pl.when(k == nsteps - 1)
    def _():
        mean = ssc_ref[...] * (1.0 / tile)
        nrm = jnp.sqrt(jnp.sum(mean * mean, axis=1, keepdims=True))
        sigs_ref[...] = mean / (nrm + 1e-8)


def _router_body(x_ref, sigs_ref, gate_ref, pos_ref, bt_ref, gsc_ref,
                 *, bn, n, nb):
    i = pl.program_id(0)
    xb = x_ref[...].astype(jnp.bfloat16)
    s = lax.dot_general(xb, sigs_ref[...].astype(jnp.bfloat16),
                        (((1,), (1,)), ((), ())),
                        preferred_element_type=jnp.float32)
    m = jnp.max(s, axis=1, keepdims=True)
    t_iota = lax.broadcasted_iota(jnp.int32, (bn, NUM_TILES), 1)
    cand = jnp.where(s == m, t_iota, NUM_TILES)
    winner = jnp.min(cand, axis=1, keepdims=True)
    gate = (t_iota == winner).astype(jnp.float32)
    gate_ref[...] = gate
    gsc_ref[pl.ds(i * bn, bn), :] = gate

    @pl.when(i == (n // bn) - 1)
    def _():
        g = gsc_ref[...]
        c = g
        sh = 1
        while sh < n:
            c = c + jnp.concatenate(
                [jnp.zeros((sh, NUM_TILES), jnp.float32), c[:-sh, :]],
                axis=0)
            sh *= 2
        rank = c - g
        counts = c[n - 1:n, :]
        pc = jnp.floor((counts + (BN - 1.0)) / BN) * BN
        p1 = pc[:, 0:1]
        p2 = p1 + pc[:, 1:2]
        p3 = p2 + pc[:, 2:3]
        pb = jnp.concatenate(
            [jnp.zeros((1, 1), jnp.float32), p1, p2, p3], axis=1)
        pos = jnp.sum(g * (rank + pb), axis=1, keepdims=True)
        pos_ref[...] = pos.astype(jnp.int32)
        ends = pb + pc
        bidx = lax.broadcasted_iota(jnp.int32, (nb, NUM_TILES), 0).astype(
            jnp.float32) * BN
        btv = jnp.sum((bidx >= ends).astype(jnp.float32), axis=1,
                      keepdims=True)
        bt_ref[...] = jnp.minimum(btv, NUM_TILES - 1.0).astype(jnp.int32)


def _ffn_body(bt_ref, xs_ref, uw_ref, usc_ref, dw_ref, dsc_ref, out_ref):
    del bt_ref
    xb = xs_ref[...].astype(jnp.bfloat16)
    h = lax.dot_general(xb, uw_ref[...], (((1,), (1,)), ((), ())),
                        preferred_element_type=jnp.float32)
    h = jnp.maximum(h * usc_ref[...], 0.0).astype(jnp.bfloat16)
    o = lax.dot_general(h, dw_ref[...], (((1,), (1,)), ((), ())),
                        preferred_element_type=jnp.float32)
    out_ref[...] = o * dsc_ref[...]


def _make_sc_scatter(n, n_pad, dp):
    per_w = n // NW
    j_s = per_w // CH
    mesh = plsc.VectorSubcoreMesh(core_axis_name="c", subcore_axis_name="s",
                                  num_cores=SC_NC, num_subcores=SC_NS)

    @functools.partial(
        pl.kernel, mesh=mesh,
        out_type=jax.ShapeDtypeStruct((n_pad, dp), jnp.float32),
        scratch_types=[
            pltpu.VMEM((CH, dp), jnp.float32),
            pltpu.VMEM((CH, dp), jnp.float32),
            pltpu.VMEM((j_s, CH), jnp.int32),
            pltpu.SemaphoreType.DMA,
            pltpu.SemaphoreType.DMA,
            pltpu.SemaphoreType.DMA,
            pltpu.SemaphoreType.DMA,
        ],
    )
    def scatter_k(x_hbm, pos3_hbm, xs_hbm, buf0, buf1, posv,
                  rs0, rs1, ws0, ws1):
        wid = lax.axis_index("s") * SC_NC + lax.axis_index("c")
        base = wid * per_w
        pltpu.sync_copy(pos3_hbm.at[wid], posv)
        bufs = (buf0, buf1)
        rsems = (rs0, rs1)
        wsems = (ws0, ws1)
        rds = [None, None]
        wrs = [None, None]
        rds[0] = pltpu.async_copy(x_hbm.at[pl.ds(base, CH)], bufs[0], rsems[0])
        for j in range(j_s):
            bsel = j & 1
            osel = bsel ^ 1
            if wrs[osel] is not None:
                wrs[osel].wait()
                wrs[osel] = None
            if j + 1 < j_s:
                rds[osel] = pltpu.async_copy(
                    x_hbm.at[pl.ds(base + (j + 1) * CH, CH)],
                    bufs[osel], rsems[osel])
            rds[bsel].wait()
            wrs[bsel] = pltpu.async_copy(
                bufs[bsel], xs_hbm.at[posv.at[j]], wsems[bsel])
        for w in wrs:
            if w is not None:
                w.wait()

    return scatter_k


def _make_sc_gather(n, n_pad, d_model):
    per_w = n // NW
    j_g = per_w // CH
    mesh = plsc.VectorSubcoreMesh(core_axis_name="c", subcore_axis_name="s",
                                  num_cores=SC_NC, num_subcores=SC_NS)

    @functools.partial(
        pl.kernel, mesh=mesh,
        out_type=jax.ShapeDtypeStruct((n, d_model), jnp.float32),
        scratch_types=[
            pltpu.VMEM((CH, d_model), jnp.float32),
            pltpu.VMEM((CH, d_model), jnp.float32),
            pltpu.VMEM((j_g, CH), jnp.int32),
            pltpu.SemaphoreType.DMA,
            pltpu.SemaphoreType.DMA,
            pltpu.SemaphoreType.DMA,
            pltpu.SemaphoreType.DMA,
        ],
    )
    def gather_k(outs_hbm, pos3_hbm, out_hbm, buf0, buf1, posv,
                 rs0, rs1, ws0, ws1):
        wid = lax.axis_index("s") * SC_NC + lax.axis_index("c")
        base = wid * per_w
        pltpu.sync_copy(pos3_hbm.at[wid], posv)
        bufs = (buf0, buf1)
        rsems = (rs0, rs1)
        wsems = (ws0, ws1)
        rds = [None, None]
        wrs = [None, None]
        rds[0] = pltpu.async_copy(outs_hbm.at[posv.at[0]], bufs[0], rsems[0])
        for j in range(j_g):
            bsel = j & 1
            osel = bsel ^ 1
            if wrs[osel] is not None:
                wrs[osel].wait()
                wrs[osel] = None
            if j + 1 < j_g:
                rds[osel] = pltpu.async_copy(
                    outs_hbm.at[posv.at[j + 1]], bufs[osel], rsems[osel])
            rds[bsel].wait()
            wrs[bsel] = pltpu.async_copy(
                bufs[bsel], out_hbm.at[pl.ds(base + j * CH, CH)],
                wsems[bsel])
        for w in wrs:
            if w is not None:
                w.wait()

    return gather_k


def kernel(x, up_w, up_scales, down_w, down_scales):
    b, t, d_model = x.shape
    d_ff = up_w.shape[0]
    tile = d_ff // NUM_TILES
    n = b * t
    n_pad = n + NUM_TILES * BN
    nb = n_pad // BN

    xf = x.reshape(n, d_model)
    usc2 = up_scales.reshape(1, d_ff)
    dsc = down_scales.reshape(1, d_model)

    rows = min(512, tile)
    nsteps = d_ff // rows
    uw2, dw2, sigs = pl.pallas_call(
        functools.partial(_prep_body, rows=rows, tile=tile, nsteps=nsteps),
        grid=(nsteps,),
        in_specs=[
            pl.BlockSpec((rows, d_model), lambda k: (k, 0)),
            pl.BlockSpec((d_model, rows), lambda k: (0, k)),
        ],
        out_specs=[
            pl.BlockSpec((rows, d_model), lambda k: (k, 0)),
            pl.BlockSpec((d_model, rows), lambda k: (0, k)),
            pl.BlockSpec((NUM_TILES, d_model), lambda k: (0, 0)),
        ],
        out_shape=[
            jax.ShapeDtypeStruct((d_ff, d_model), jnp.bfloat16),
            jax.ShapeDtypeStruct((d_model, d_ff), jnp.bfloat16),
            jax.ShapeDtypeStruct((NUM_TILES, d_model), jnp.float32),
        ],
        scratch_shapes=[pltpu.VMEM((NUM_TILES, d_model), jnp.float32)],
        compiler_params=pltpu.CompilerParams(
            dimension_semantics=("arbitrary",),
        ),
    )(up_w, down_w)

    bn_r = min(512, n)
    gate, pos, bt = pl.pallas_call(
        functools.partial(_router_body, bn=bn_r, n=n, nb=nb),
        grid=(n // bn_r,),
        in_specs=[
            pl.BlockSpec((bn_r, d_model), lambda i: (i, 0)),
            pl.BlockSpec((NUM_TILES, d_model), lambda i: (0, 0)),
        ],
        out_specs=[
            pl.BlockSpec((bn_r, NUM_TILES), lambda i: (i, 0)),
            pl.BlockSpec((n, 1), lambda i: (0, 0)),
            pl.BlockSpec((nb, 1), lambda i: (0, 0)),
        ],
        out_shape=[
            jax.ShapeDtypeStruct((n, NUM_TILES), jnp.float32),
            jax.ShapeDtypeStruct((n, 1), jnp.int32),
            jax.ShapeDtypeStruct((nb, 1), jnp.int32),
        ],
        scratch_shapes=[pltpu.VMEM((n, NUM_TILES), jnp.float32)],
    )(xf, sigs)
    posf = pos.reshape(n)
    btf = bt.reshape(nb)

    pos3 = posf.reshape(NW, n // NW // CH, CH)
    xs = _make_sc_scatter(n, n_pad, d_model)(xf, pos3)

    grid_spec = pltpu.PrefetchScalarGridSpec(
        num_scalar_prefetch=1,
        grid=(nb,),
        in_specs=[
            pl.BlockSpec((BN, d_model), lambda i, bt_r: (i, 0)),
            pl.BlockSpec((tile, d_model), lambda i, bt_r: (bt_r[i], 0)),
            pl.BlockSpec((1, tile), lambda i, bt_r: (0, bt_r[i])),
            pl.BlockSpec((d_model, tile), lambda i, bt_r: (0, bt_r[i])),
            pl.BlockSpec((1, d_model), lambda i, bt_r: (0, 0)),
        ],
        out_specs=pl.BlockSpec((BN, d_model), lambda i, bt_r: (i, 0)),
    )
    outs = pl.pallas_call(
        _ffn_body,
        grid_spec=grid_spec,
        out_shape=jax.ShapeDtypeStruct((n_pad, d_model), jnp.float32),
        compiler_params=pltpu.CompilerParams(
            dimension_semantics=("arbitrary",),
        ),
    )(btf, xs, uw2, usc2, dw2, dsc)

    out = _make_sc_gather(n, n_pad, d_model)(outs, pos3)

    return out.reshape(b, t, d_model), gate.reshape(b, t, NUM_TILES)
